# baseline (speedup 1.0000x reference)
import functools

import jax
import jax.numpy as jnp
from jax import lax
from jax.experimental import pallas as pl
from jax.experimental.pallas import tpu as pltpu
from jax.experimental.pallas import tpu_sc as plsc

OUT = 7
S = 14
NBOX = 512
C = 256
BASE0 = 0
BASE1 = 2 * 256 * 256
BASE2 = BASE1 + 2 * 128 * 128
BASE3 = BASE2 + 2 * 64 * 64
NROWS = BASE3 + 2 * 32 * 32


def _prep_body(bx_ref, idx_ref, w_ref):
    f32 = jnp.float32
    x1 = bx_ref[0:1, :]
    y1 = bx_ref[1:2, :]
    x2 = bx_ref[2:3, :]
    y2 = bx_ref[3:4, :]
    area = (x2 - x1) * (y2 - y1)
    v = jnp.sqrt(area) / 224.0 + 1e-8
    lvl = ((v >= 0.5).astype(jnp.int32) + (v >= 1.0).astype(jnp.int32)
           + (v >= 2.0).astype(jnp.int32))
    scale = jnp.where(lvl == 0, f32(0.25),
             jnp.where(lvl == 1, f32(0.125),
              jnp.where(lvl == 2, f32(0.0625), f32(0.03125))))
    Hn = jnp.where(lvl == 0, 256, jnp.where(lvl == 1, 128,
          jnp.where(lvl == 2, 64, 32)))
    base = jnp.where(lvl == 0, BASE0, jnp.where(lvl == 1, BASE1,
            jnp.where(lvl == 2, BASE2, BASE3)))
    Hf = Hn.astype(f32)
    x1s = x1 * scale - 0.5
    y1s = y1 * scale - 0.5
    bw = (x2 * scale - 0.5 - x1s) / f32(OUT)
    bh = (y2 * scale - 0.5 - y1s) / f32(OUT)

    s = lax.broadcasted_iota(jnp.int32, (S * S, NBOX), 0)
    box = lax.broadcasted_iota(jnp.int32, (S * S, NBOX), 1)
    sy = s // S
    sx = s - sy * S
    yy = y1s + (sy.astype(f32) * 0.5 + 0.25) * bh
    xx = x1s + (sx.astype(f32) * 0.5 + 0.25) * bw
    valid = ((yy >= -1.0) & (yy <= Hf)) & ((xx >= -1.0) & (xx <= Hf))
    y = jnp.maximum(yy, 0.0)
    x = jnp.maximum(xx, 0.0)
    y0 = jnp.floor(y).astype(jnp.int32)
    x0 = jnp.floor(x).astype(jnp.int32)
    ych = y0 >= Hn - 1
    xch = x0 >= Hn - 1
    y_low = jnp.where(ych, Hn - 1, y0)
    y_high = jnp.where(ych, Hn - 1, y0 + 1)
    yc = jnp.where(ych, Hf - 1.0, y)
    x_low = jnp.where(xch, Hn - 1, x0)
    x_high = jnp.where(xch, Hn - 1, x0 + 1)
    xc = jnp.where(xch, Hf - 1.0, x)
    ly = yc - y_low.astype(f32)
    lx = xc - x_low.astype(f32)
    hy = 1.0 - ly
    hx = 1.0 - lx
    vm = jnp.where(valid, f32(0.25), f32(0.0))

    b = box // 256
    rowbase = base + b * (Hn * Hn)
    idx_ref[0] = rowbase + y_low * Hn + x_low
    idx_ref[1] = rowbase + y_low * Hn + x_high
    idx_ref[2] = rowbase + y_high * Hn + x_low
    idx_ref[3] = rowbase + y_high * Hn + x_high
    w_ref[0] = hy * hx * vm
    w_ref[1] = hy * lx * vm
    w_ref[2] = ly * hx * vm
    w_ref[3] = ly * lx * vm


_prep = pl.pallas_call(
    _prep_body,
    out_shape=[
        jax.ShapeDtypeStruct((4, S * S, NBOX), jnp.int32),
        jax.ShapeDtypeStruct((4, S * S, NBOX), jnp.float32),
    ],
)


def _lane_bcast(vec, j):
    return lax.gather(
        vec,
        jnp.full((16, 1), j, jnp.int32),
        lax.GatherDimensionNumbers(
            offset_dims=(), collapsed_slice_dims=(0,), start_index_map=(0,)),
        (1,),
        mode=lax.GatherScatterMode.PROMISE_IN_BOUNDS,
    )


def _sc_pool(table, idx3, wflat):
    mesh = plsc.VectorSubcoreMesh(core_axis_name="c", subcore_axis_name="s")

    @functools.partial(
        pl.kernel,
        mesh=mesh,
        out_type=jax.ShapeDtypeStruct((NBOX, 49, C), jnp.float32),
        scratch_types=[
            pltpu.VMEM((7, 112), jnp.int32),
            pltpu.VMEM((784,), jnp.float32),
            pltpu.VMEM((2, 112, C // 2), jnp.int32),
            pltpu.VMEM((49, C), jnp.float32),
            pltpu.SemaphoreType.DMA,
            pltpu.SemaphoreType.DMA,
        ],
        compiler_params=pltpu.CompilerParams(needs_layout_passes=False),
    )
    def k(table_hbm, idx_hbm, w_hbm, out_hbm, idx_v, w_v, rows_v, acc_v,
          sem0, sem1):
        wid = lax.axis_index("s") * 2 + lax.axis_index("c")
        sems = [sem0, sem1]

        def box_body(t, carry):
            bi = wid * 16 + t
            pltpu.sync_copy(idx_hbm.at[bi], idx_v)
            pltpu.sync_copy(w_hbm.at[bi], w_v)

            cps = [None, None]
            cps[0] = pltpu.async_copy(
                table_hbm.at[idx_v.at[0]], rows_v.at[0], sems[0])
            for c in range(7):
                b = c % 2
                if c + 1 < 7:
                    nb = (c + 1) % 2
                    cps[nb] = pltpu.async_copy(
                        table_hbm.at[idx_v.at[c + 1]], rows_v.at[nb], sems[nb])
                cps[b].wait()

                def px_body(p, carry3, c=c, b=b):
                    off = c * 112 + p * 16
                    wvec = w_v[pl.ds(off, 16)]
                    acc = [jnp.zeros((16,), jnp.float32)
                           for _ in range(C // 16)]
                    for j in range(16):
                        wj = _lane_bcast(wvec, j)
                        r = p * 16 + j
                        for cc in range(C // 32):
                            pw = rows_v[b, r, pl.ds(cc * 16, 16)]
                            bf = plsc.bitcast(pw, jnp.bfloat16)
                            lo, hi = plsc.unpack(
                                bf, format=plsc.PackFormat.INTERLEAVED)
                            acc[2 * cc] = acc[2 * cc] + wj * lo
                            acc[2 * cc + 1] = acc[2 * cc + 1] + wj * hi
                    pg = c * 7 + p
                    for cc in range(C // 16):
                        acc_v[pg, pl.ds(cc * 16, 16)] = acc[cc]
                    return carry3

                lax.fori_loop(0, 7, px_body, 0)
            pltpu.sync_copy(acc_v, out_hbm.at[bi])
            return carry

        lax.fori_loop(0, 16, box_body, 0)

    return k(table, idx3, wflat)


def kernel(x_p2, x_p3, x_p4, x_p5, boxes):
    def pack(x):
        t = x.astype(jnp.bfloat16).transpose(0, 2, 3, 1)
        return lax.bitcast_convert_type(
            t.reshape(-1, C // 2, 2), jnp.int32)

    table = jnp.concatenate(
        [pack(x_p2), pack(x_p3), pack(x_p4), pack(x_p5)], axis=0)

    bxs = boxes.reshape(NBOX, 4).T
    idx4, w4 = _prep(bxs)

    def reorder(a):
        a = a.transpose(2, 1, 0).reshape(NBOX, OUT, 2, OUT, 2, 4)
        return a.transpose(0, 1, 3, 2, 4, 5).reshape(NBOX, 49 * 16)

    idx = reorder(idx4).reshape(NBOX, 7, 112)
    w = reorder(w4)

    out = _sc_pool(table, idx, w)
    out = out.reshape(NBOX, 49, 8, 2, 16).transpose(0, 2, 4, 3, 1)
    return out.reshape(NBOX, C, OUT, OUT)


# speedup vs baseline: 1.3894x; 1.3894x over previous
import functools

import jax
import jax.numpy as jnp
from jax import lax
from jax.experimental import pallas as pl
from jax.experimental.pallas import tpu as pltpu
from jax.experimental.pallas import tpu_sc as plsc

OUT = 7
S = 14
NBOX = 512
C = 256
N = 2
LEVEL_H = (256, 128, 64, 32)


def _pack_body(x_ref, o_ref):
    lo = x_ref[0, :, 0, :]
    hi = x_ref[0, :, 1, :]
    lo16 = lax.bitcast_convert_type(
        lo.astype(jnp.bfloat16), jnp.uint16).astype(jnp.int32)
    hi16 = lax.bitcast_convert_type(
        hi.astype(jnp.bfloat16), jnp.uint16).astype(jnp.int32)
    word = lo16 | (hi16 << 16)
    o_ref[...] = word.T


def _pack_level(x, H):
    th = min(H, max(8, 2048 // H))
    P = th * H
    nblk = H // th
    xr = x.reshape(N, C // 2, 2, H * H)
    return pl.pallas_call(
        _pack_body,
        grid=(N, nblk),
        in_specs=[pl.BlockSpec((1, C // 2, 2, P), lambda n, r: (n, 0, 0, r))],
        out_specs=pl.BlockSpec((P, C // 2), lambda n, r: (n * nblk + r, 0)),
        out_shape=jax.ShapeDtypeStruct((N * H * H, C // 2), jnp.int32),
    )(xr)


def _prep_body(bx_ref, idx_ref, w_ref, lvl_ref):
    f32 = jnp.float32
    x1 = bx_ref[:, 0:1]
    y1 = bx_ref[:, 1:2]
    x2 = bx_ref[:, 2:3]
    y2 = bx_ref[:, 3:4]
    area = (x2 - x1) * (y2 - y1)
    v = jnp.sqrt(area) / 224.0 + 1e-8
    lvl = ((v >= 0.5).astype(jnp.int32) + (v >= 1.0).astype(jnp.int32)
           + (v >= 2.0).astype(jnp.int32))
    scale = jnp.where(lvl == 0, f32(0.25),
             jnp.where(lvl == 1, f32(0.125),
              jnp.where(lvl == 2, f32(0.0625), f32(0.03125))))
    Hn = jnp.where(lvl == 0, 256, jnp.where(lvl == 1, 128,
          jnp.where(lvl == 2, 64, 32)))
    Hf = Hn.astype(f32)
    x1s = x1 * scale - 0.5
    y1s = y1 * scale - 0.5
    bw = (x2 * scale - 0.5 - x1s) / f32(OUT)
    bh = (y2 * scale - 0.5 - y1s) / f32(OUT)

    j = lax.broadcasted_iota(jnp.int32, (NBOX, 784), 1)
    box = lax.broadcasted_iota(jnp.int32, (NBOX, 784), 0)
    p = j // 16
    py = p // OUT
    px = p - py * OUT
    r = j - p * 16
    uy = r // 8
    ux = (r - uy * 8) // 4
    corner = r - uy * 8 - ux * 4
    sy = 2 * py + uy
    sx = 2 * px + ux
    yy = y1s + (sy.astype(f32) * 0.5 + 0.25) * bh
    xx = x1s + (sx.astype(f32) * 0.5 + 0.25) * bw
    valid = ((yy >= -1.0) & (yy <= Hf)) & ((xx >= -1.0) & (xx <= Hf))
    y = jnp.maximum(yy, 0.0)
    x = jnp.maximum(xx, 0.0)
    y0 = jnp.floor(y).astype(jnp.int32)
    x0 = jnp.floor(x).astype(jnp.int32)
    ych = y0 >= Hn - 1
    xch = x0 >= Hn - 1
    y_low = jnp.where(ych, Hn - 1, y0)
    y_high = jnp.where(ych, Hn - 1, y0 + 1)
    yc = jnp.where(ych, Hf - 1.0, y)
    x_low = jnp.where(xch, Hn - 1, x0)
    x_high = jnp.where(xch, Hn - 1, x0 + 1)
    xc = jnp.where(xch, Hf - 1.0, x)
    ly = yc - y_low.astype(f32)
    lx = xc - x_low.astype(f32)
    hy = 1.0 - ly
    hx = 1.0 - lx
    vm = jnp.where(valid, f32(0.25), f32(0.0))

    ysel = corner >= 2
    xsel = (corner - (corner // 2) * 2) == 1
    cy = jnp.where(ysel, y_high, y_low)
    cx = jnp.where(xsel, x_high, x_low)
    wy = jnp.where(ysel, ly, hy)
    wx = jnp.where(xsel, lx, hx)

    b = box // 256
    idx_ref[...] = b * (Hn * Hn) + cy * Hn + cx
    w_ref[...] = wy * wx * vm
    lvl_ref[...] = lvl


_prep = pl.pallas_call(
    _prep_body,
    out_shape=[
        jax.ShapeDtypeStruct((NBOX, 784), jnp.int32),
        jax.ShapeDtypeStruct((NBOX, 784), jnp.float32),
        jax.ShapeDtypeStruct((NBOX, 1), jnp.int32),
    ],
)


def _lane_bcast(vec, j):
    return lax.gather(
        vec,
        jnp.full((16, 1), j, jnp.int32),
        lax.GatherDimensionNumbers(
            offset_dims=(), collapsed_slice_dims=(0,), start_index_map=(0,)),
        (1,),
        mode=lax.GatherScatterMode.PROMISE_IN_BOUNDS,
    )


def _sc_pool(t0, t1, t2, t3, idx3, wflat, lvl):
    mesh = plsc.VectorSubcoreMesh(core_axis_name="c", subcore_axis_name="s")

    @functools.partial(
        pl.kernel,
        mesh=mesh,
        out_type=jax.ShapeDtypeStruct((NBOX, 49, C), jnp.float32),
        scratch_types=[
            pltpu.VMEM((16,), jnp.int32),
            pltpu.VMEM((7, 112), jnp.int32),
            pltpu.VMEM((784,), jnp.float32),
            pltpu.VMEM((2, 112, C // 2), jnp.int32),
            pltpu.VMEM((49, C), jnp.float32),
            pltpu.SemaphoreType.DMA,
            pltpu.SemaphoreType.DMA,
        ],
        compiler_params=pltpu.CompilerParams(needs_layout_passes=False),
    )
    def k(t0_hbm, t1_hbm, t2_hbm, t3_hbm, idx_hbm, w_hbm, lvl_hbm, out_hbm,
          lvl_v, idx_v, w_v, rows_v, acc_v, sem0, sem1):
        wid = lax.axis_index("s") * 2 + lax.axis_index("c")
        tabs = [t0_hbm, t1_hbm, t2_hbm, t3_hbm]
        sems = [sem0, sem1]
        pltpu.sync_copy(lvl_hbm.at[pl.ds(wid * 16, 16)], lvl_v)
        lvl16 = lvl_v[...]

        def box_body(t, carry):
            bi = wid * 16 + t
            lv = jnp.max(_lane_bcast(lvl16, t))
            pltpu.sync_copy(idx_hbm.at[bi], idx_v)
            pltpu.sync_copy(w_hbm.at[bi], w_v)

            def issue(c, nb):
                for L in range(4):
                    @pl.when(lv == L)
                    def _(L=L, c=c, nb=nb):
                        pltpu.async_copy(
                            tabs[L].at[idx_v.at[c]], rows_v.at[nb], sems[nb])

            issue(0, 0)
            for c in range(7):
                b = c % 2
                if c + 1 < 7:
                    issue(c + 1, (c + 1) % 2)
                pltpu.make_async_copy(
                    tabs[0].at[idx_v.at[c]], rows_v.at[b], sems[b]).wait()

                def px_body(p, carry3, c=c, b=b):
                    off = c * 112 + p * 16
                    wvec = w_v[pl.ds(off, 16)]
                    acc = [jnp.zeros((16,), jnp.float32)
                           for _ in range(C // 16)]
                    for j in range(16):
                        wj = _lane_bcast(wvec, j)
                        r = p * 16 + j
                        for cc in range(C // 32):
                            pw = rows_v[b, r, pl.ds(cc * 16, 16)]
                            bf = plsc.bitcast(pw, jnp.bfloat16)
                            lo, hi = plsc.unpack(
                                bf, format=plsc.PackFormat.INTERLEAVED)
                            acc[2 * cc] = acc[2 * cc] + wj * lo
                            acc[2 * cc + 1] = acc[2 * cc + 1] + wj * hi
                    pg = c * 7 + p
                    for cc in range(C // 16):
                        acc_v[pg, pl.ds(cc * 16, 16)] = acc[cc]
                    return carry3

                lax.fori_loop(0, 7, px_body, 0)
            pltpu.sync_copy(acc_v, out_hbm.at[bi])
            return carry

        lax.fori_loop(0, 16, box_body, 0)

    return k(t0, t1, t2, t3, idx3, wflat, lvl)


def kernel(x_p2, x_p3, x_p4, x_p5, boxes):
    tables = [_pack_level(x, H) for x, H in
              zip((x_p2, x_p3, x_p4, x_p5), LEVEL_H)]

    idx, w, lvl = _prep(boxes.reshape(NBOX, 4))
    out = _sc_pool(*tables, idx.reshape(NBOX, 7, 112), w, lvl.reshape(NBOX))

    out = out.reshape(NBOX, 49, 8, 2, 16).transpose(0, 2, 4, 3, 1)
    return out.reshape(NBOX, C, OUT, OUT)


# speedup vs baseline: 1.7993x; 1.2950x over previous
import functools

import jax
import jax.numpy as jnp
from jax import lax
from jax.experimental import pallas as pl
from jax.experimental.pallas import tpu as pltpu
from jax.experimental.pallas import tpu_sc as plsc

OUT = 7
S = 14
NBOX = 512
C = 256
N = 2
LEVEL_H = (256, 128, 64, 32)


def _pack_level(x, H):
    lo16 = lax.bitcast_convert_type(
        x[:, 0::2].astype(jnp.bfloat16), jnp.uint16).astype(jnp.int32)
    hi16 = lax.bitcast_convert_type(
        x[:, 1::2].astype(jnp.bfloat16), jnp.uint16).astype(jnp.int32)
    word = lax.optimization_barrier(lo16 | (hi16 << 16))
    return word.transpose(0, 2, 3, 1).reshape(N * H * H, C // 2)


def _prep_body(bx_ref, idx_ref, w_ref, lvl_ref):
    f32 = jnp.float32
    x1 = bx_ref[:, 0:1]
    y1 = bx_ref[:, 1:2]
    x2 = bx_ref[:, 2:3]
    y2 = bx_ref[:, 3:4]
    area = (x2 - x1) * (y2 - y1)
    v = jnp.sqrt(area) / 224.0 + 1e-8
    lvl = ((v >= 0.5).astype(jnp.int32) + (v >= 1.0).astype(jnp.int32)
           + (v >= 2.0).astype(jnp.int32))
    scale = jnp.where(lvl == 0, f32(0.25),
             jnp.where(lvl == 1, f32(0.125),
              jnp.where(lvl == 2, f32(0.0625), f32(0.03125))))
    Hn = jnp.where(lvl == 0, 256, jnp.where(lvl == 1, 128,
          jnp.where(lvl == 2, 64, 32)))
    Hf = Hn.astype(f32)
    x1s = x1 * scale - 0.5
    y1s = y1 * scale - 0.5
    bw = (x2 * scale - 0.5 - x1s) / f32(OUT)
    bh = (y2 * scale - 0.5 - y1s) / f32(OUT)

    j = lax.broadcasted_iota(jnp.int32, (NBOX, 784), 1)
    box = lax.broadcasted_iota(jnp.int32, (NBOX, 784), 0)
    p = j // 16
    py = p // OUT
    px = p - py * OUT
    r = j - p * 16
    uy = r // 8
    ux = (r - uy * 8) // 4
    corner = r - uy * 8 - ux * 4
    sy = 2 * py + uy
    sx = 2 * px + ux
    yy = y1s + (sy.astype(f32) * 0.5 + 0.25) * bh
    xx = x1s + (sx.astype(f32) * 0.5 + 0.25) * bw
    valid = ((yy >= -1.0) & (yy <= Hf)) & ((xx >= -1.0) & (xx <= Hf))
    y = jnp.maximum(yy, 0.0)
    x = jnp.maximum(xx, 0.0)
    y0 = jnp.floor(y).astype(jnp.int32)
    x0 = jnp.floor(x).astype(jnp.int32)
    ych = y0 >= Hn - 1
    xch = x0 >= Hn - 1
    y_low = jnp.where(ych, Hn - 1, y0)
    y_high = jnp.where(ych, Hn - 1, y0 + 1)
    yc = jnp.where(ych, Hf - 1.0, y)
    x_low = jnp.where(xch, Hn - 1, x0)
    x_high = jnp.where(xch, Hn - 1, x0 + 1)
    xc = jnp.where(xch, Hf - 1.0, x)
    ly = yc - y_low.astype(f32)
    lx = xc - x_low.astype(f32)
    hy = 1.0 - ly
    hx = 1.0 - lx
    vm = jnp.where(valid, f32(0.25), f32(0.0))

    ysel = corner >= 2
    xsel = (corner - (corner // 2) * 2) == 1
    cy = jnp.where(ysel, y_high, y_low)
    cx = jnp.where(xsel, x_high, x_low)
    wy = jnp.where(ysel, ly, hy)
    wx = jnp.where(xsel, lx, hx)

    b = box // 256
    idx_ref[...] = b * (Hn * Hn) + cy * Hn + cx
    w_ref[...] = wy * wx * vm
    lvl_ref[...] = lvl


_prep = pl.pallas_call(
    _prep_body,
    out_shape=[
        jax.ShapeDtypeStruct((NBOX, 784), jnp.int32),
        jax.ShapeDtypeStruct((NBOX, 784), jnp.float32),
        jax.ShapeDtypeStruct((NBOX, 1), jnp.int32),
    ],
)


def _lane_bcast(vec, j):
    return lax.gather(
        vec,
        jnp.full((16, 1), j, jnp.int32),
        lax.GatherDimensionNumbers(
            offset_dims=(), collapsed_slice_dims=(0,), start_index_map=(0,)),
        (1,),
        mode=lax.GatherScatterMode.PROMISE_IN_BOUNDS,
    )


def _sc_pool(t0, t1, t2, t3, idxflat, wflat, lvl):
    mesh = plsc.VectorSubcoreMesh(core_axis_name="c", subcore_axis_name="s")

    @functools.partial(
        pl.kernel,
        mesh=mesh,
        out_type=jax.ShapeDtypeStruct((NBOX, 49, C), jnp.float32),
        scratch_types=[
            pltpu.VMEM((16,), jnp.int32),
            pltpu.VMEM((784,), jnp.int32),
            pltpu.VMEM((784,), jnp.float32),
            pltpu.VMEM((2, 112, C // 2), jnp.int32),
            pltpu.VMEM((49, C), jnp.float32),
            pltpu.SemaphoreType.DMA,
            pltpu.SemaphoreType.DMA,
        ],
        compiler_params=pltpu.CompilerParams(needs_layout_passes=False),
    )
    def k(t0_hbm, t1_hbm, t2_hbm, t3_hbm, idx_hbm, w_hbm, lvl_hbm, out_hbm,
          lvl_v, idx_v, w_v, rows_v, acc_v, sem0, sem1):
        wid = lax.axis_index("s") * 2 + lax.axis_index("c")
        tabs = [t0_hbm, t1_hbm, t2_hbm, t3_hbm]
        sems = [sem0, sem1]
        pltpu.sync_copy(lvl_hbm.at[pl.ds(wid * 16, 16)], lvl_v)
        lvl16 = lvl_v[...]

        def box_body(t, carry):
            bi = wid * 16 + t
            lv = jnp.max(_lane_bcast(lvl16, t))
            pltpu.sync_copy(idx_hbm.at[bi], idx_v)
            pltpu.sync_copy(w_hbm.at[bi], w_v)

            def issue(c, nb):
                for L in range(4):
                    @pl.when(lv == L)
                    def _(L=L, c=c, nb=nb):
                        pltpu.async_copy(
                            tabs[L].at[idx_v.at[pl.ds(c * 112, 112)]],
                            rows_v.at[nb], sems[nb])

            issue(0, 0)
            for c in range(7):
                b = c % 2
                if c + 1 < 7:
                    issue(c + 1, (c + 1) % 2)
                pltpu.make_async_copy(
                    tabs[0].at[idx_v.at[pl.ds(c * 112, 112)]],
                    rows_v.at[b], sems[b]).wait()

                def px_body(p, carry3, c=c, b=b):
                    off = c * 112 + p * 16
                    wvec = w_v[pl.ds(off, 16)]
                    acc = [jnp.zeros((16,), jnp.float32)
                           for _ in range(C // 16)]
                    for j in range(16):
                        wj = _lane_bcast(wvec, j)
                        r = p * 16 + j
                        for cc in range(C // 32):
                            pw = rows_v[b, r, pl.ds(cc * 16, 16)]
                            bf = plsc.bitcast(pw, jnp.bfloat16)
                            lo, hi = plsc.unpack(
                                bf, format=plsc.PackFormat.INTERLEAVED)
                            acc[2 * cc] = acc[2 * cc] + wj * lo
                            acc[2 * cc + 1] = acc[2 * cc + 1] + wj * hi
                    pg = c * 7 + p
                    for cc in range(C // 16):
                        acc_v[pg, pl.ds(cc * 16, 16)] = acc[cc]
                    return carry3

                lax.fori_loop(0, 7, px_body, 0)
            pltpu.sync_copy(acc_v, out_hbm.at[bi])
            return carry

        lax.fori_loop(0, 16, box_body, 0)

    return k(t0, t1, t2, t3, idxflat, wflat, lvl)


def kernel(x_p2, x_p3, x_p4, x_p5, boxes):
    tables = [_pack_level(x, H) for x, H in
              zip((x_p2, x_p3, x_p4, x_p5), LEVEL_H)]

    idx, w, lvl = _prep(boxes.reshape(NBOX, 4))
    out = _sc_pool(*tables, idx, w, lvl.reshape(NBOX))

    out = out.reshape(NBOX, 49, 8, 2, 16).transpose(0, 2, 4, 3, 1)
    return out.reshape(NBOX, C, OUT, OUT)


# speedup vs baseline: 1.9540x; 1.0860x over previous
import functools

import jax
import jax.numpy as jnp
from jax import lax
from jax.experimental import pallas as pl
from jax.experimental.pallas import tpu as pltpu
from jax.experimental.pallas import tpu_sc as plsc

OUT = 7
S = 14
NBOX = 512
C = 256
N = 2
LEVEL_H = (256, 128, 64, 32)


def _pack_level(x, H):
    u = lax.bitcast_convert_type(
        x.astype(jnp.bfloat16), jnp.uint16).astype(jnp.int32)
    odd = lax.broadcasted_iota(jnp.int32, (1, C, 1, 1), 1) & 1
    word = (u << (odd * 16)).reshape(N, C // 2, 2, H, H).sum(axis=2)
    word = lax.optimization_barrier(word)
    return word.transpose(0, 2, 3, 1).reshape(N * H * H, C // 2)


def _prep_body(bx_ref, idx_ref, w_ref, lvl_ref):
    f32 = jnp.float32
    x1 = bx_ref[:, 0:1]
    y1 = bx_ref[:, 1:2]
    x2 = bx_ref[:, 2:3]
    y2 = bx_ref[:, 3:4]
    area = (x2 - x1) * (y2 - y1)
    v = jnp.sqrt(area) / 224.0 + 1e-8
    lvl = ((v >= 0.5).astype(jnp.int32) + (v >= 1.0).astype(jnp.int32)
           + (v >= 2.0).astype(jnp.int32))
    scale = jnp.where(lvl == 0, f32(0.25),
             jnp.where(lvl == 1, f32(0.125),
              jnp.where(lvl == 2, f32(0.0625), f32(0.03125))))
    Hn = jnp.where(lvl == 0, 256, jnp.where(lvl == 1, 128,
          jnp.where(lvl == 2, 64, 32)))
    Hf = Hn.astype(f32)
    x1s = x1 * scale - 0.5
    y1s = y1 * scale - 0.5
    bw = (x2 * scale - 0.5 - x1s) / f32(OUT)
    bh = (y2 * scale - 0.5 - y1s) / f32(OUT)

    j = lax.broadcasted_iota(jnp.int32, (NBOX, 784), 1)
    box = lax.broadcasted_iota(jnp.int32, (NBOX, 784), 0)
    p = j // 16
    py = p // OUT
    px = p - py * OUT
    r = j - p * 16
    uy = r // 8
    ux = (r - uy * 8) // 4
    corner = r - uy * 8 - ux * 4
    sy = 2 * py + uy
    sx = 2 * px + ux
    yy = y1s + (sy.astype(f32) * 0.5 + 0.25) * bh
    xx = x1s + (sx.astype(f32) * 0.5 + 0.25) * bw
    valid = ((yy >= -1.0) & (yy <= Hf)) & ((xx >= -1.0) & (xx <= Hf))
    y = jnp.maximum(yy, 0.0)
    x = jnp.maximum(xx, 0.0)
    y0 = jnp.floor(y).astype(jnp.int32)
    x0 = jnp.floor(x).astype(jnp.int32)
    ych = y0 >= Hn - 1
    xch = x0 >= Hn - 1
    y_low = jnp.where(ych, Hn - 1, y0)
    y_high = jnp.where(ych, Hn - 1, y0 + 1)
    yc = jnp.where(ych, Hf - 1.0, y)
    x_low = jnp.where(xch, Hn - 1, x0)
    x_high = jnp.where(xch, Hn - 1, x0 + 1)
    xc = jnp.where(xch, Hf - 1.0, x)
    ly = yc - y_low.astype(f32)
    lx = xc - x_low.astype(f32)
    hy = 1.0 - ly
    hx = 1.0 - lx
    vm = jnp.where(valid, f32(0.25), f32(0.0))

    ysel = corner >= 2
    xsel = (corner - (corner // 2) * 2) == 1
    cy = jnp.where(ysel, y_high, y_low)
    cx = jnp.where(xsel, x_high, x_low)
    wy = jnp.where(ysel, ly, hy)
    wx = jnp.where(xsel, lx, hx)

    b = box // 256
    idx_ref[...] = b * (Hn * Hn) + cy * Hn + cx
    w_ref[...] = wy * wx * vm
    lvl_ref[...] = lvl


_prep = pl.pallas_call(
    _prep_body,
    out_shape=[
        jax.ShapeDtypeStruct((NBOX, 784), jnp.int32),
        jax.ShapeDtypeStruct((NBOX, 784), jnp.float32),
        jax.ShapeDtypeStruct((NBOX, 1), jnp.int32),
    ],
)


def _lane_bcast(vec, j):
    return lax.gather(
        vec,
        jnp.full((16, 1), j, jnp.int32),
        lax.GatherDimensionNumbers(
            offset_dims=(), collapsed_slice_dims=(0,), start_index_map=(0,)),
        (1,),
        mode=lax.GatherScatterMode.PROMISE_IN_BOUNDS,
    )


def _sc_pool(t0, t1, t2, t3, idxflat, wflat, lvl):
    mesh = plsc.VectorSubcoreMesh(core_axis_name="c", subcore_axis_name="s")

    @functools.partial(
        pl.kernel,
        mesh=mesh,
        out_type=jax.ShapeDtypeStruct((NBOX, 49, C), jnp.float32),
        scratch_types=[
            pltpu.VMEM((16,), jnp.int32),
            pltpu.VMEM((784,), jnp.int32),
            pltpu.VMEM((784,), jnp.float32),
            pltpu.VMEM((2, 112, C // 2), jnp.int32),
            pltpu.VMEM((49, C), jnp.float32),
            pltpu.SemaphoreType.DMA,
            pltpu.SemaphoreType.DMA,
        ],
        compiler_params=pltpu.CompilerParams(needs_layout_passes=False),
    )
    def k(t0_hbm, t1_hbm, t2_hbm, t3_hbm, idx_hbm, w_hbm, lvl_hbm, out_hbm,
          lvl_v, idx_v, w_v, rows_v, acc_v, sem0, sem1):
        wid = lax.axis_index("s") * 2 + lax.axis_index("c")
        tabs = [t0_hbm, t1_hbm, t2_hbm, t3_hbm]
        sems = [sem0, sem1]
        pltpu.sync_copy(lvl_hbm.at[pl.ds(wid * 16, 16)], lvl_v)
        lvl16 = lvl_v[...]

        def box_body(t, carry):
            bi = wid * 16 + t
            lv = jnp.max(_lane_bcast(lvl16, t))
            pltpu.sync_copy(idx_hbm.at[bi], idx_v)
            pltpu.sync_copy(w_hbm.at[bi], w_v)

            def issue(c, nb):
                for L in range(4):
                    @pl.when(lv == L)
                    def _(L=L, c=c, nb=nb):
                        pltpu.async_copy(
                            tabs[L].at[idx_v.at[pl.ds(c * 112, 112)]],
                            rows_v.at[nb], sems[nb])

            issue(0, 0)
            for c in range(7):
                b = c % 2
                if c + 1 < 7:
                    issue(c + 1, (c + 1) % 2)
                pltpu.make_async_copy(
                    tabs[0].at[idx_v.at[pl.ds(c * 112, 112)]],
                    rows_v.at[b], sems[b]).wait()

                def px_body(p, carry3, c=c, b=b):
                    off = c * 112 + p * 16
                    wvec = w_v[pl.ds(off, 16)]
                    acc = [jnp.zeros((16,), jnp.float32)
                           for _ in range(C // 16)]
                    for j in range(16):
                        wj = _lane_bcast(wvec, j)
                        r = p * 16 + j
                        for cc in range(C // 32):
                            pw = rows_v[b, r, pl.ds(cc * 16, 16)]
                            bf = plsc.bitcast(pw, jnp.bfloat16)
                            lo, hi = plsc.unpack(
                                bf, format=plsc.PackFormat.INTERLEAVED)
                            acc[2 * cc] = acc[2 * cc] + wj * lo
                            acc[2 * cc + 1] = acc[2 * cc + 1] + wj * hi
                    pg = c * 7 + p
                    for cc in range(C // 16):
                        acc_v[pg, pl.ds(cc * 16, 16)] = acc[cc]
                    return carry3

                lax.fori_loop(0, 7, px_body, 0)
            pltpu.sync_copy(acc_v, out_hbm.at[bi])
            return carry

        lax.fori_loop(0, 16, box_body, 0)

    return k(t0, t1, t2, t3, idxflat, wflat, lvl)


def kernel(x_p2, x_p3, x_p4, x_p5, boxes):
    tables = [_pack_level(x, H) for x, H in
              zip((x_p2, x_p3, x_p4, x_p5), LEVEL_H)]

    idx, w, lvl = _prep(boxes.reshape(NBOX, 4))
    out = _sc_pool(*tables, idx, w, lvl.reshape(NBOX))

    out = out.reshape(NBOX, 49, 8, 2, 16).transpose(0, 2, 4, 3, 1)
    return out.reshape(NBOX, C, OUT, OUT)


# speedup vs baseline: 2.0162x; 1.0318x over previous
import functools

import jax
import jax.numpy as jnp
from jax import lax
from jax.experimental import pallas as pl
from jax.experimental.pallas import tpu as pltpu
from jax.experimental.pallas import tpu_sc as plsc

OUT = 7
S = 14
NBOX = 512
C = 256
N = 2
LEVEL_H = (256, 128, 64, 32)


def _pack_level(x, H):
    xr = x.reshape(N, C // 2, 2, H, H)
    lo16 = lax.bitcast_convert_type(
        xr[:, :, 0].astype(jnp.bfloat16), jnp.uint16).astype(jnp.int32)
    hi16 = lax.bitcast_convert_type(
        xr[:, :, 1].astype(jnp.bfloat16), jnp.uint16).astype(jnp.int32)
    word = lax.optimization_barrier(lo16 | (hi16 << 16))
    return word.transpose(0, 2, 3, 1).reshape(N * H * H, C // 2)


def _prep_body(bx_ref, idx_ref, w_ref, lvl_ref):
    f32 = jnp.float32
    x1 = bx_ref[:, 0:1]
    y1 = bx_ref[:, 1:2]
    x2 = bx_ref[:, 2:3]
    y2 = bx_ref[:, 3:4]
    area = (x2 - x1) * (y2 - y1)
    v = jnp.sqrt(area) / 224.0 + 1e-8
    lvl = ((v >= 0.5).astype(jnp.int32) + (v >= 1.0).astype(jnp.int32)
           + (v >= 2.0).astype(jnp.int32))
    scale = jnp.where(lvl == 0, f32(0.25),
             jnp.where(lvl == 1, f32(0.125),
              jnp.where(lvl == 2, f32(0.0625), f32(0.03125))))
    Hn = jnp.where(lvl == 0, 256, jnp.where(lvl == 1, 128,
          jnp.where(lvl == 2, 64, 32)))
    Hf = Hn.astype(f32)
    x1s = x1 * scale - 0.5
    y1s = y1 * scale - 0.5
    bw = (x2 * scale - 0.5 - x1s) / f32(OUT)
    bh = (y2 * scale - 0.5 - y1s) / f32(OUT)

    j = lax.broadcasted_iota(jnp.int32, (NBOX, 784), 1)
    box = lax.broadcasted_iota(jnp.int32, (NBOX, 784), 0)
    p = j // 16
    py = p // OUT
    px = p - py * OUT
    r = j - p * 16
    uy = r // 8
    ux = (r - uy * 8) // 4
    corner = r - uy * 8 - ux * 4
    sy = 2 * py + uy
    sx = 2 * px + ux
    yy = y1s + (sy.astype(f32) * 0.5 + 0.25) * bh
    xx = x1s + (sx.astype(f32) * 0.5 + 0.25) * bw
    valid = ((yy >= -1.0) & (yy <= Hf)) & ((xx >= -1.0) & (xx <= Hf))
    y = jnp.maximum(yy, 0.0)
    x = jnp.maximum(xx, 0.0)
    y0 = jnp.floor(y).astype(jnp.int32)
    x0 = jnp.floor(x).astype(jnp.int32)
    ych = y0 >= Hn - 1
    xch = x0 >= Hn - 1
    y_low = jnp.where(ych, Hn - 1, y0)
    y_high = jnp.where(ych, Hn - 1, y0 + 1)
    yc = jnp.where(ych, Hf - 1.0, y)
    x_low = jnp.where(xch, Hn - 1, x0)
    x_high = jnp.where(xch, Hn - 1, x0 + 1)
    xc = jnp.where(xch, Hf - 1.0, x)
    ly = yc - y_low.astype(f32)
    lx = xc - x_low.astype(f32)
    hy = 1.0 - ly
    hx = 1.0 - lx
    vm = jnp.where(valid, f32(0.25), f32(0.0))

    ysel = corner >= 2
    xsel = (corner - (corner // 2) * 2) == 1
    cy = jnp.where(ysel, y_high, y_low)
    cx = jnp.where(xsel, x_high, x_low)
    wy = jnp.where(ysel, ly, hy)
    wx = jnp.where(xsel, lx, hx)

    b = box // 256
    idx_ref[...] = b * (Hn * Hn) + cy * Hn + cx
    w_ref[...] = wy * wx * vm
    lvl_ref[...] = lvl


_prep = pl.pallas_call(
    _prep_body,
    out_shape=[
        jax.ShapeDtypeStruct((NBOX, 784), jnp.int32),
        jax.ShapeDtypeStruct((NBOX, 784), jnp.float32),
        jax.ShapeDtypeStruct((NBOX, 1), jnp.int32),
    ],
)


def _lane_bcast(vec, j):
    return lax.gather(
        vec,
        jnp.full((16, 1), j, jnp.int32),
        lax.GatherDimensionNumbers(
            offset_dims=(), collapsed_slice_dims=(0,), start_index_map=(0,)),
        (1,),
        mode=lax.GatherScatterMode.PROMISE_IN_BOUNDS,
    )


def _sc_pool(t0, t1, t2, t3, idxflat, wflat, lvl):
    mesh = plsc.VectorSubcoreMesh(core_axis_name="c", subcore_axis_name="s")

    @functools.partial(
        pl.kernel,
        mesh=mesh,
        out_type=jax.ShapeDtypeStruct((NBOX, 49, C), jnp.float32),
        scratch_types=[
            pltpu.VMEM((16,), jnp.int32),
            pltpu.VMEM((784,), jnp.int32),
            pltpu.VMEM((784,), jnp.float32),
            pltpu.VMEM((2, 112, C // 2), jnp.int32),
            pltpu.VMEM((49, C), jnp.float32),
            pltpu.SemaphoreType.DMA,
            pltpu.SemaphoreType.DMA,
        ],
        compiler_params=pltpu.CompilerParams(needs_layout_passes=False),
    )
    def k(t0_hbm, t1_hbm, t2_hbm, t3_hbm, idx_hbm, w_hbm, lvl_hbm, out_hbm,
          lvl_v, idx_v, w_v, rows_v, acc_v, sem0, sem1):
        wid = lax.axis_index("s") * 2 + lax.axis_index("c")
        tabs = [t0_hbm, t1_hbm, t2_hbm, t3_hbm]
        sems = [sem0, sem1]
        pltpu.sync_copy(lvl_hbm.at[pl.ds(wid * 16, 16)], lvl_v)
        lvl16 = lvl_v[...]

        def box_body(t, carry):
            bi = wid * 16 + t
            lv = jnp.max(_lane_bcast(lvl16, t))
            pltpu.sync_copy(idx_hbm.at[bi], idx_v)
            pltpu.sync_copy(w_hbm.at[bi], w_v)

            def issue(c, nb):
                for L in range(4):
                    @pl.when(lv == L)
                    def _(L=L, c=c, nb=nb):
                        pltpu.async_copy(
                            tabs[L].at[idx_v.at[pl.ds(c * 112, 112)]],
                            rows_v.at[nb], sems[nb])

            issue(0, 0)
            for c in range(7):
                b = c % 2
                if c + 1 < 7:
                    issue(c + 1, (c + 1) % 2)
                pltpu.make_async_copy(
                    tabs[0].at[idx_v.at[pl.ds(c * 112, 112)]],
                    rows_v.at[b], sems[b]).wait()

                def px_body(p, carry3, c=c, b=b):
                    off = c * 112 + p * 16
                    wvec = w_v[pl.ds(off, 16)]
                    acc = [jnp.zeros((16,), jnp.float32)
                           for _ in range(C // 16)]
                    for j in range(16):
                        wj = _lane_bcast(wvec, j)
                        r = p * 16 + j
                        for cc in range(C // 32):
                            pw = rows_v[b, r, pl.ds(cc * 16, 16)]
                            bf = plsc.bitcast(pw, jnp.bfloat16)
                            lo, hi = plsc.unpack(
                                bf, format=plsc.PackFormat.INTERLEAVED)
                            acc[2 * cc] = acc[2 * cc] + wj * lo
                            acc[2 * cc + 1] = acc[2 * cc + 1] + wj * hi
                    pg = c * 7 + p
                    rowi = jnp.full((16,), pg, jnp.int32)
                    two_iota = lax.broadcasted_iota(jnp.int32, (16,), 0) * 2
                    for cc in range(C // 32):
                        coli = two_iota + (cc * 32)
                        plsc.store_scatter(acc_v, [rowi, coli], acc[2 * cc])
                        plsc.store_scatter(
                            acc_v, [rowi, coli + 1], acc[2 * cc + 1])
                    return carry3

                lax.fori_loop(0, 7, px_body, 0)
            pltpu.sync_copy(acc_v, out_hbm.at[bi])
            return carry

        lax.fori_loop(0, 16, box_body, 0)

    return k(t0, t1, t2, t3, idxflat, wflat, lvl)


def kernel(x_p2, x_p3, x_p4, x_p5, boxes):
    tables = [_pack_level(x, H) for x, H in
              zip((x_p2, x_p3, x_p4, x_p5), LEVEL_H)]

    idx, w, lvl = _prep(boxes.reshape(NBOX, 4))
    out = _sc_pool(*tables, idx, w, lvl.reshape(NBOX))

    return out.transpose(0, 2, 1).reshape(NBOX, C, OUT, OUT)


# speedup vs baseline: 2.6968x; 1.3376x over previous
import functools

import jax
import jax.numpy as jnp
from jax import lax
from jax.experimental import pallas as pl
from jax.experimental.pallas import tpu as pltpu
from jax.experimental.pallas import tpu_sc as plsc

OUT = 7
S = 14
NBOX = 512
C = 256
N = 2
LEVEL_H = (256, 128, 64, 32)


def _pack_level(x, H):
    lo16 = lax.bitcast_convert_type(
        x[:, :C // 2].astype(jnp.bfloat16), jnp.uint16).astype(jnp.int32)
    hi16 = lax.bitcast_convert_type(
        x[:, C // 2:].astype(jnp.bfloat16), jnp.uint16).astype(jnp.int32)
    word = lax.optimization_barrier(lo16 | (hi16 << 16))
    return word.transpose(0, 2, 3, 1).reshape(N * H * H, C // 2)


def _prep_body(bx_ref, idx_ref, w_ref, lvl_ref):
    f32 = jnp.float32
    x1 = bx_ref[:, 0:1]
    y1 = bx_ref[:, 1:2]
    x2 = bx_ref[:, 2:3]
    y2 = bx_ref[:, 3:4]
    area = (x2 - x1) * (y2 - y1)
    v = jnp.sqrt(area) / 224.0 + 1e-8
    lvl = ((v >= 0.5).astype(jnp.int32) + (v >= 1.0).astype(jnp.int32)
           + (v >= 2.0).astype(jnp.int32))
    scale = jnp.where(lvl == 0, f32(0.25),
             jnp.where(lvl == 1, f32(0.125),
              jnp.where(lvl == 2, f32(0.0625), f32(0.03125))))
    Hn = jnp.where(lvl == 0, 256, jnp.where(lvl == 1, 128,
          jnp.where(lvl == 2, 64, 32)))
    Hf = Hn.astype(f32)
    x1s = x1 * scale - 0.5
    y1s = y1 * scale - 0.5
    bw = (x2 * scale - 0.5 - x1s) / f32(OUT)
    bh = (y2 * scale - 0.5 - y1s) / f32(OUT)

    j = lax.broadcasted_iota(jnp.int32, (NBOX, 784), 1)
    box = lax.broadcasted_iota(jnp.int32, (NBOX, 784), 0)
    p = j // 16
    py = p // OUT
    px = p - py * OUT
    r = j - p * 16
    uy = r // 8
    ux = (r - uy * 8) // 4
    corner = r - uy * 8 - ux * 4
    sy = 2 * py + uy
    sx = 2 * px + ux
    yy = y1s + (sy.astype(f32) * 0.5 + 0.25) * bh
    xx = x1s + (sx.astype(f32) * 0.5 + 0.25) * bw
    valid = ((yy >= -1.0) & (yy <= Hf)) & ((xx >= -1.0) & (xx <= Hf))
    y = jnp.maximum(yy, 0.0)
    x = jnp.maximum(xx, 0.0)
    y0 = jnp.floor(y).astype(jnp.int32)
    x0 = jnp.floor(x).astype(jnp.int32)
    ych = y0 >= Hn - 1
    xch = x0 >= Hn - 1
    y_low = jnp.where(ych, Hn - 1, y0)
    y_high = jnp.where(ych, Hn - 1, y0 + 1)
    yc = jnp.where(ych, Hf - 1.0, y)
    x_low = jnp.where(xch, Hn - 1, x0)
    x_high = jnp.where(xch, Hn - 1, x0 + 1)
    xc = jnp.where(xch, Hf - 1.0, x)
    ly = yc - y_low.astype(f32)
    lx = xc - x_low.astype(f32)
    hy = 1.0 - ly
    hx = 1.0 - lx
    vm = jnp.where(valid, f32(0.25), f32(0.0))

    ysel = corner >= 2
    xsel = (corner - (corner // 2) * 2) == 1
    cy = jnp.where(ysel, y_high, y_low)
    cx = jnp.where(xsel, x_high, x_low)
    wy = jnp.where(ysel, ly, hy)
    wx = jnp.where(xsel, lx, hx)

    b = box // 256
    idx_ref[...] = b * (Hn * Hn) + cy * Hn + cx
    w_ref[...] = wy * wx * vm
    lvl_ref[...] = lvl


_prep = pl.pallas_call(
    _prep_body,
    out_shape=[
        jax.ShapeDtypeStruct((NBOX, 784), jnp.int32),
        jax.ShapeDtypeStruct((NBOX, 784), jnp.float32),
        jax.ShapeDtypeStruct((NBOX, 1), jnp.int32),
    ],
)


def _lane_bcast(vec, j):
    return lax.gather(
        vec,
        jnp.full((16, 1), j, jnp.int32),
        lax.GatherDimensionNumbers(
            offset_dims=(), collapsed_slice_dims=(0,), start_index_map=(0,)),
        (1,),
        mode=lax.GatherScatterMode.PROMISE_IN_BOUNDS,
    )


def _sc_pool(t0, t1, t2, t3, idxflat, wflat, lvl):
    mesh = plsc.VectorSubcoreMesh(core_axis_name="c", subcore_axis_name="s")

    @functools.partial(
        pl.kernel,
        mesh=mesh,
        out_type=jax.ShapeDtypeStruct((NBOX, 49, C), jnp.float32),
        scratch_types=[
            pltpu.VMEM((16,), jnp.int32),
            pltpu.VMEM((784,), jnp.int32),
            pltpu.VMEM((784,), jnp.float32),
            pltpu.VMEM((2, 112, C // 2), jnp.int32),
            pltpu.VMEM((49, C), jnp.float32),
            pltpu.SemaphoreType.DMA,
            pltpu.SemaphoreType.DMA,
        ],
        compiler_params=pltpu.CompilerParams(needs_layout_passes=False),
    )
    def k(t0_hbm, t1_hbm, t2_hbm, t3_hbm, idx_hbm, w_hbm, lvl_hbm, out_hbm,
          lvl_v, idx_v, w_v, rows_v, acc_v, sem0, sem1):
        wid = lax.axis_index("s") * 2 + lax.axis_index("c")
        tabs = [t0_hbm, t1_hbm, t2_hbm, t3_hbm]
        sems = [sem0, sem1]
        pltpu.sync_copy(lvl_hbm.at[pl.ds(wid * 16, 16)], lvl_v)
        lvl16 = lvl_v[...]

        def box_body(t, carry):
            bi = wid * 16 + t
            lv = jnp.max(_lane_bcast(lvl16, t))
            pltpu.sync_copy(idx_hbm.at[bi], idx_v)
            pltpu.sync_copy(w_hbm.at[bi], w_v)

            def issue(c, nb):
                for L in range(4):
                    @pl.when(lv == L)
                    def _(L=L, c=c, nb=nb):
                        pltpu.async_copy(
                            tabs[L].at[idx_v.at[pl.ds(c * 112, 112)]],
                            rows_v.at[nb], sems[nb])

            issue(0, 0)
            for c in range(7):
                b = c % 2
                if c + 1 < 7:
                    issue(c + 1, (c + 1) % 2)
                pltpu.make_async_copy(
                    tabs[0].at[idx_v.at[pl.ds(c * 112, 112)]],
                    rows_v.at[b], sems[b]).wait()

                def px_body(p, carry3, c=c, b=b):
                    off = c * 112 + p * 16
                    wvec = w_v[pl.ds(off, 16)]
                    acc = [jnp.zeros((16,), jnp.float32)
                           for _ in range(C // 16)]
                    for j in range(16):
                        wj = _lane_bcast(wvec, j)
                        r = p * 16 + j
                        for cc in range(C // 32):
                            pw = rows_v[b, r, pl.ds(cc * 16, 16)]
                            bf = plsc.bitcast(pw, jnp.bfloat16)
                            lo, hi = plsc.unpack(
                                bf, format=plsc.PackFormat.INTERLEAVED)
                            acc[cc] = acc[cc] + wj * lo
                            acc[cc + 8] = acc[cc + 8] + wj * hi
                    pg = c * 7 + p
                    for cc in range(C // 16):
                        acc_v[pg, pl.ds(cc * 16, 16)] = acc[cc]
                    return carry3

                lax.fori_loop(0, 7, px_body, 0)
            pltpu.sync_copy(acc_v, out_hbm.at[bi])
            return carry

        lax.fori_loop(0, 16, box_body, 0)

    return k(t0, t1, t2, t3, idxflat, wflat, lvl)


def kernel(x_p2, x_p3, x_p4, x_p5, boxes):
    tables = [_pack_level(x, H) for x, H in
              zip((x_p2, x_p3, x_p4, x_p5), LEVEL_H)]

    idx, w, lvl = _prep(boxes.reshape(NBOX, 4))
    out = _sc_pool(*tables, idx, w, lvl.reshape(NBOX))

    return out.transpose(0, 2, 1).reshape(NBOX, C, OUT, OUT)
